# baseline (speedup 1.0000x reference)
import functools

import jax
import jax.numpy as jnp
from jax import lax
from jax.experimental import pallas as pl
from jax.experimental.pallas import tpu as pltpu
from jax.experimental.pallas import tpu_sc as plsc

DIM = 768
N_ART = 32 * 512
N_Q = 32 * 64
N_OPT = 32 * 5 * 48

NUM_CORES = 2
NUM_SUBCORES = 16
NW = NUM_CORES * NUM_SUBCORES


def _sc_gather(word_emb, ids, row_off, rows_per_w, chunk):
    n_rows = rows_per_w * NW
    n = rows_per_w // chunk
    depth = min(2, n)
    mesh = plsc.VectorSubcoreMesh(core_axis_name="c", subcore_axis_name="s")

    @functools.partial(
        pl.kernel,
        mesh=mesh,
        out_type=jax.ShapeDtypeStruct((n_rows, DIM), jnp.float32),
        scratch_types=(
            [pltpu.VMEM((rows_per_w,), jnp.int32)]
            + [pltpu.VMEM((chunk, DIM), jnp.float32)] * depth
            + [pltpu.SemaphoreType.DMA] * (2 * depth)
        ),
    )
    def k(table_hbm, idx_hbm, out_hbm, idx_v, *bufs_sems):
        bufs = bufs_sems[:depth]
        gsems = bufs_sems[depth:2 * depth]
        wsems = bufs_sems[2 * depth:]
        wid = lax.axis_index("s") * NUM_CORES + lax.axis_index("c")
        base = wid * rows_per_w
        pltpu.sync_copy(idx_hbm.at[pl.ds(row_off + base, rows_per_w)], idx_v)

        def gather(c):
            cp = pltpu.make_async_copy(
                table_hbm.at[idx_v.at[pl.ds(c * chunk, chunk)]],
                bufs[c % depth], gsems[c % depth])
            cp.start()
            return cp

        def write(c):
            cp = pltpu.make_async_copy(
                bufs[c % depth],
                out_hbm.at[pl.ds(base + c * chunk, chunk)],
                wsems[c % depth])
            cp.start()
            return cp

        gathers = [gather(c0) for c0 in range(max(1, depth - 1))]
        writes = []
        for c in range(n):
            gathers[c].wait()
            writes.append(write(c))
            nxt = c + max(1, depth - 1)
            if nxt < n and nxt > len(gathers) - 1:
                if nxt - depth >= 0:
                    writes[nxt - depth].wait()
                gathers.append(gather(nxt))
        for c in range(max(0, n - depth), n):
            writes[c].wait()

    return k(word_emb, ids)


def _ln_body(g_ref, add_ref, gam_ref, bet_ref, o_ref):
    rows = g_ref.shape[0]
    add = add_ref[...]
    if add.shape[0] != rows:
        reps = rows // add.shape[0]
        add = jnp.broadcast_to(add[None], (reps,) + add.shape).reshape(rows,
                                                                       DIM)
    x = g_ref[...] + add
    mu = jnp.mean(x, axis=1, keepdims=True)
    xc = x - mu
    var = jnp.mean(xc * xc, axis=1, keepdims=True)
    o_ref[...] = xc * lax.rsqrt(var + 1e-12) * gam_ref[...] + bet_ref[...]


def _ln_alias_body(g_ref, _old_ref, add_ref, gam_ref, bet_ref, o_ref):
    _ln_body(g_ref, add_ref, gam_ref, bet_ref, o_ref)


def _ln_call(gathered, addtab, gamma2d, beta2d, block, out_rows=None):
    nrows = gathered.shape[0]
    grid = nrows // block
    add_rows = addtab.shape[0]
    if out_rows is None:
        out_rows = nrows
    return pl.pallas_call(
        _ln_body,
        grid=(grid,),
        in_specs=[
            pl.BlockSpec((block, DIM), lambda i: (i, 0)),
            pl.BlockSpec((add_rows, DIM), lambda i: (0, 0)),
            pl.BlockSpec((1, DIM), lambda i: (0, 0)),
            pl.BlockSpec((1, DIM), lambda i: (0, 0)),
        ],
        out_specs=pl.BlockSpec((block, DIM), lambda i: (i, 0)),
        out_shape=jax.ShapeDtypeStruct((out_rows, DIM), jnp.float32),
    )(gathered, addtab, gamma2d, beta2d)


def _ln_call_alias(gathered, partial_out, addtab, gamma2d, beta2d, block,
                   blk_off):
    nrows = gathered.shape[0]
    grid = nrows // block
    add_rows = addtab.shape[0]
    return pl.pallas_call(
        _ln_alias_body,
        grid=(grid,),
        in_specs=[
            pl.BlockSpec((block, DIM), lambda i: (i, 0)),
            pl.BlockSpec((8, 128), lambda i: (0, 0)),
            pl.BlockSpec((add_rows, DIM), lambda i: (0, 0)),
            pl.BlockSpec((1, DIM), lambda i: (0, 0)),
            pl.BlockSpec((1, DIM), lambda i: (0, 0)),
        ],
        out_specs=pl.BlockSpec((block, DIM), lambda i: (i + blk_off, 0)),
        out_shape=jax.ShapeDtypeStruct(partial_out.shape, jnp.float32),
        input_output_aliases={1: 0},
    )(gathered, partial_out, addtab, gamma2d, beta2d)


def kernel(article_tokens, question_tokens, options_tokens, word_emb,
           pos_emb, tok_type_emb, gamma, beta):
    art_ids = article_tokens.reshape(-1).astype(jnp.int32)
    q_ids = question_tokens.reshape(-1).astype(jnp.int32)
    opt_ids = options_tokens.reshape(-1).astype(jnp.int32)
    half = N_ART // 2

    goh = _sc_gather(word_emb, opt_ids, 0, rows_per_w=80, chunk=40)
    ga1 = _sc_gather(word_emb, art_ids, 0, rows_per_w=256, chunk=64)
    gor = _sc_gather(word_emb, opt_ids, 2560, rows_per_w=160, chunk=40)
    ga2 = _sc_gather(word_emb, art_ids, half, rows_per_w=256, chunk=64)
    gq = _sc_gather(word_emb, q_ids, 0, rows_per_w=64, chunk=64)

    addvec = pos_emb + tok_type_emb[0]
    q_add = addvec[:64]
    o_add = addvec[:1]
    g2 = gamma.reshape(1, DIM)
    b2 = beta.reshape(1, DIM)

    art1 = _ln_call(ga1, addvec, g2, b2, block=512, out_rows=N_ART)
    opt1 = _ln_call(goh, o_add, g2, b2, block=512, out_rows=N_OPT)
    opt = _ln_call_alias(gor, opt1, o_add, g2, b2, block=512, blk_off=5)
    art = _ln_call_alias(ga2, art1, addvec, g2, b2, block=512,
                         blk_off=half // 512)
    q = _ln_call(gq, q_add, g2, b2, block=512)

    return (art.reshape(32, 512, DIM),
            q.reshape(32, 64, DIM),
            opt.reshape(32, 5, 48, DIM))


# speedup vs baseline: 1.0183x; 1.0183x over previous
import functools

import jax
import jax.numpy as jnp
from jax import lax
from jax.experimental import pallas as pl
from jax.experimental.pallas import tpu as pltpu
from jax.experimental.pallas import tpu_sc as plsc

DIM = 768
N_ART = 32 * 512
N_Q = 32 * 64
N_OPT = 32 * 5 * 48

NUM_CORES = 2
NUM_SUBCORES = 16
NW = NUM_CORES * NUM_SUBCORES


def _sc_gather(word_emb, ids, row_off, rows_per_w, chunk):
    n_rows = rows_per_w * NW
    n = rows_per_w // chunk
    depth = min(2, n)
    mesh = plsc.VectorSubcoreMesh(core_axis_name="c", subcore_axis_name="s")

    @functools.partial(
        pl.kernel,
        mesh=mesh,
        out_type=jax.ShapeDtypeStruct((n_rows, DIM), jnp.float32),
        scratch_types=(
            [pltpu.VMEM((rows_per_w,), jnp.int32)]
            + [pltpu.VMEM((chunk, DIM), jnp.float32)] * depth
            + [pltpu.SemaphoreType.DMA] * (2 * depth)
        ),
    )
    def k(table_hbm, idx_hbm, out_hbm, idx_v, *bufs_sems):
        bufs = bufs_sems[:depth]
        gsems = bufs_sems[depth:2 * depth]
        wsems = bufs_sems[2 * depth:]
        wid = lax.axis_index("s") * NUM_CORES + lax.axis_index("c")
        base = wid * rows_per_w
        pltpu.sync_copy(idx_hbm.at[pl.ds(row_off + base, rows_per_w)], idx_v)

        def gather(c):
            cp = pltpu.make_async_copy(
                table_hbm.at[idx_v.at[pl.ds(c * chunk, chunk)]],
                bufs[c % depth], gsems[c % depth])
            cp.start()
            return cp

        def write(c):
            cp = pltpu.make_async_copy(
                bufs[c % depth],
                out_hbm.at[pl.ds(base + c * chunk, chunk)],
                wsems[c % depth])
            cp.start()
            return cp

        gathers = [gather(c0) for c0 in range(max(1, depth - 1))]
        writes = []
        for c in range(n):
            nxt = c + max(1, depth - 1)
            if nxt < n and nxt > len(gathers) - 1:
                if nxt - depth >= 0:
                    writes[nxt - depth].wait()
                gathers.append(gather(nxt))
            gathers[c].wait()
            writes.append(write(c))
        for c in range(max(0, n - depth), n):
            writes[c].wait()

    return k(word_emb, ids)


def _ln_body(g_ref, add_ref, gam_ref, bet_ref, o_ref):
    rows = g_ref.shape[0]
    add = add_ref[...]
    if add.shape[0] != rows:
        reps = rows // add.shape[0]
        add = jnp.broadcast_to(add[None], (reps,) + add.shape).reshape(rows,
                                                                       DIM)
    x = g_ref[...] + add
    mu = jnp.mean(x, axis=1, keepdims=True)
    xc = x - mu
    var = jnp.mean(xc * xc, axis=1, keepdims=True)
    o_ref[...] = xc * lax.rsqrt(var + 1e-12) * gam_ref[...] + bet_ref[...]


def _ln_alias_body(g_ref, _old_ref, add_ref, gam_ref, bet_ref, o_ref):
    _ln_body(g_ref, add_ref, gam_ref, bet_ref, o_ref)


def _ln_call(gathered, addtab, gamma2d, beta2d, block, out_rows=None):
    nrows = gathered.shape[0]
    grid = nrows // block
    add_rows = addtab.shape[0]
    if out_rows is None:
        out_rows = nrows
    return pl.pallas_call(
        _ln_body,
        grid=(grid,),
        in_specs=[
            pl.BlockSpec((block, DIM), lambda i: (i, 0)),
            pl.BlockSpec((add_rows, DIM), lambda i: (0, 0)),
            pl.BlockSpec((1, DIM), lambda i: (0, 0)),
            pl.BlockSpec((1, DIM), lambda i: (0, 0)),
        ],
        out_specs=pl.BlockSpec((block, DIM), lambda i: (i, 0)),
        out_shape=jax.ShapeDtypeStruct((out_rows, DIM), jnp.float32),
    )(gathered, addtab, gamma2d, beta2d)


def _ln_call_alias(gathered, partial_out, addtab, gamma2d, beta2d, block,
                   blk_off):
    nrows = gathered.shape[0]
    grid = nrows // block
    add_rows = addtab.shape[0]
    return pl.pallas_call(
        _ln_alias_body,
        grid=(grid,),
        in_specs=[
            pl.BlockSpec((block, DIM), lambda i: (i, 0)),
            pl.BlockSpec((8, 128), lambda i: (0, 0)),
            pl.BlockSpec((add_rows, DIM), lambda i: (0, 0)),
            pl.BlockSpec((1, DIM), lambda i: (0, 0)),
            pl.BlockSpec((1, DIM), lambda i: (0, 0)),
        ],
        out_specs=pl.BlockSpec((block, DIM), lambda i: (i + blk_off, 0)),
        out_shape=jax.ShapeDtypeStruct(partial_out.shape, jnp.float32),
        input_output_aliases={1: 0},
    )(gathered, partial_out, addtab, gamma2d, beta2d)


def kernel(article_tokens, question_tokens, options_tokens, word_emb,
           pos_emb, tok_type_emb, gamma, beta):
    art_ids = article_tokens.reshape(-1).astype(jnp.int32)
    q_ids = question_tokens.reshape(-1).astype(jnp.int32)
    opt_ids = options_tokens.reshape(-1).astype(jnp.int32)
    half = N_ART // 2

    goh = _sc_gather(word_emb, opt_ids, 0, rows_per_w=80, chunk=40)
    ga1 = _sc_gather(word_emb, art_ids, 0, rows_per_w=256, chunk=64)
    gor = _sc_gather(word_emb, opt_ids, 2560, rows_per_w=160, chunk=40)
    ga2 = _sc_gather(word_emb, art_ids, half, rows_per_w=256, chunk=64)
    gq = _sc_gather(word_emb, q_ids, 0, rows_per_w=64, chunk=64)

    addvec = pos_emb + tok_type_emb[0]
    q_add = addvec[:64]
    o_add = addvec[:1]
    g2 = gamma.reshape(1, DIM)
    b2 = beta.reshape(1, DIM)

    art1 = _ln_call(ga1, addvec, g2, b2, block=512, out_rows=N_ART)
    opt1 = _ln_call(goh, o_add, g2, b2, block=512, out_rows=N_OPT)
    opt = _ln_call_alias(gor, opt1, o_add, g2, b2, block=512, blk_off=5)
    art = _ln_call_alias(ga2, art1, addvec, g2, b2, block=512,
                         blk_off=half // 512)
    q = _ln_call(gq, q_add, g2, b2, block=512)

    return (art.reshape(32, 512, DIM),
            q.reshape(32, 64, DIM),
            opt.reshape(32, 5, 48, DIM))
